# baseline (speedup 1.0000x reference)
import functools
import types

import jax
import jax.numpy as jnp
from jax import lax
from jax.experimental import pallas as pl
from jax.experimental.pallas import tpu as pltpu
from jax.experimental.pallas import tpu_sc as plsc

N = 10000
E = 160000
B = 512
HN = 256
HE = 64
HG = 32
L = 3

EP = E // 2
NP = 10240
NC = 2
NS = 16
NW = NC * NS
EPW = E // NW
GCH = 200
NCH = EPW // GCH
NPW = NP // NW
NSB = 640
F32 = jnp.float32


def _lin_relu_body(x, w, b, o):
    o[...] = jnp.maximum(
        jnp.dot(x[...], w[...], preferred_element_type=F32) + b[...], 0.0)


def _lin_relu(x, w, b, rows):
    m, k = x.shape
    n = w.shape[1]
    return pl.pallas_call(
        _lin_relu_body,
        grid=(m // rows,),
        in_specs=[pl.BlockSpec((rows, k), lambda i: (i, 0)),
                  pl.BlockSpec((k, n), lambda i: (0, 0)),
                  pl.BlockSpec((1, n), lambda i: (0, 0))],
        out_specs=pl.BlockSpec((rows, n), lambda i: (i, 0)),
        out_shape=jax.ShapeDtypeStruct((m, n), F32),
    )(x, w, b)


def _encT_body(x, w, b, o):
    o[...] = jnp.maximum(
        lax.dot_general(x[...], w[...], (((0,), (0,)), ((), ())),
                        preferred_element_type=F32) + b[...], 0.0)


def _encT(xt, w, b, rows):
    k, m = xt.shape
    n = w.shape[1]
    return pl.pallas_call(
        _encT_body,
        grid=(m // rows,),
        in_specs=[pl.BlockSpec((k, rows), lambda i: (0, i)),
                  pl.BlockSpec((k, n), lambda i: (0, 0)),
                  pl.BlockSpec((1, n), lambda i: (0, 0))],
        out_specs=pl.BlockSpec((rows, n), lambda i: (i, 0)),
        out_shape=jax.ShapeDtypeStruct((m, n), F32),
    )(xt, w, b)


def _nprep_body(x, ws, wn, ns_o, nw_o):
    xv = x[...]
    t = jnp.dot(xv, ws[...], preferred_element_type=F32)
    ns_o[...] = jnp.concatenate([t, t], axis=1)
    nw_o[...] = jnp.dot(xv, wn[...], preferred_element_type=F32)


def _nprep(nmat, ws, wn, rows):
    return pl.pallas_call(
        _nprep_body,
        grid=(NP // rows,),
        in_specs=[pl.BlockSpec((rows, HN), lambda i: (i, 0)),
                  pl.BlockSpec((HN, HE), lambda i: (0, 0)),
                  pl.BlockSpec((HN, HN), lambda i: (0, 0))],
        out_specs=[pl.BlockSpec((rows, 2 * HE), lambda i: (i, 0)),
                   pl.BlockSpec((rows, HN), lambda i: (i, 0))],
        out_shape=[jax.ShapeDtypeStruct((NP, 2 * HE), F32),
                   jax.ShapeDtypeStruct((NP, HN), F32)],
    )(nmat, ws, wn)


def _enew_body(e, g, w, b, o):
    o[...] = jnp.maximum(
        jnp.dot(e[...], w[...], preferred_element_type=F32) + g[...] + b[...],
        0.0)


def _enew(e, gns, w, b, rows):
    return pl.pallas_call(
        _enew_body,
        grid=(EP // rows,),
        in_specs=[pl.BlockSpec((rows, 2 * HE), lambda i: (i, 0)),
                  pl.BlockSpec((rows, 2 * HE), lambda i: (i, 0)),
                  pl.BlockSpec((2 * HE, 2 * HE), lambda i: (0, 0)),
                  pl.BlockSpec((1, 2 * HE), lambda i: (0, 0))],
        out_specs=pl.BlockSpec((rows, 2 * HE), lambda i: (i, 0)),
        out_shape=jax.ShapeDtypeStruct((EP, 2 * HE), F32),
    )(e, gns, w, b)


def _nnew_body(nw, a0, a1, d0, d1, win, b, o):
    agg = (a0[...] + a1[...]) / jnp.maximum(d0[...] + d1[...], 1.0)
    o[...] = jnp.maximum(
        nw[...] + jnp.dot(agg, win[...], preferred_element_type=F32) + b[...],
        0.0)


def _nnew(nw, a0, a1, d0, d1, win, b, rows):
    return pl.pallas_call(
        _nnew_body,
        grid=(NP // rows,),
        in_specs=[pl.BlockSpec((rows, HN), lambda i: (i, 0)),
                  pl.BlockSpec((rows, HE), lambda i: (i, 0)),
                  pl.BlockSpec((rows, HE), lambda i: (i, 0)),
                  pl.BlockSpec((rows, 1), lambda i: (i, 0)),
                  pl.BlockSpec((rows, 1), lambda i: (i, 0)),
                  pl.BlockSpec((HE, HN), lambda i: (0, 0)),
                  pl.BlockSpec((1, HN), lambda i: (0, 0))],
        out_specs=pl.BlockSpec((rows, HN), lambda i: (i, 0)),
        out_shape=jax.ShapeDtypeStruct((NP, HN), F32),
    )(nw, a0, a1, d0, d1, win, b)


def _globchain_body(*refs):
    (np0, np1, ep0, ep1, nc0, nc1, ec0, ec1,
     wn0, we0, b0, wns, wes, wgs, bgs, row, rob, o) = refs
    inv_n = 1.0 / jnp.maximum(nc0[...] + nc1[...], 1.0)
    inv_e = 1.0 / jnp.maximum(ec0[...] + ec1[...], 1.0)

    def stage(i, g, wn, we, b):
        nmean = (np0[i] + np1[i]) * inv_n
        emean = (ep0[i] + ep1[i]) * inv_e
        acc = (jnp.dot(nmean, wn, preferred_element_type=F32)
               + jnp.dot(emean, we, preferred_element_type=F32) + b)
        if g is not None:
            acc = acc + jnp.dot(g, wgs[i - 1], preferred_element_type=F32)
        return jnp.maximum(acc, 0.0)

    g = stage(0, None, wn0[...], we0[...], b0[...])
    for i in range(1, L + 1):
        g = stage(i, g, wns[i - 1], wes[i - 1], bgs[i - 1])
    o[...] = jnp.dot(g, row[...], preferred_element_type=F32) + rob[...]


def _globchain(np0, np1, ep0, ep1, nc0, nc1, ec0, ec1,
               wn0, we0, b0, wns, wes, wgs, bgs, row, rob):
    return pl.pallas_call(
        _globchain_body,
        out_shape=jax.ShapeDtypeStruct((B, 1), F32),
    )(np0, np1, ep0, ep1, nc0, nc1, ec0, ec1,
      wn0, we0, b0, wns, wes, wgs, bgs, row, rob)


@functools.cache
def _sc():
    mesh = plsc.VectorSubcoreMesh(core_axis_name="c", subcore_axis_name="s",
                                  num_cores=NC, num_subcores=NS)
    params = pltpu.CompilerParams(use_tc_tiling_on_sc=False)

    @functools.partial(
        pl.kernel, mesh=mesh, compiler_params=params,
        out_type=jax.ShapeDtypeStruct((E, HE), F32),
        scratch_types=[pltpu.VMEM((NCH, GCH), jnp.int32),
                       pltpu.VMEM((GCH, HE), F32),
                       pltpu.VMEM((GCH, HE), F32),
                       pltpu.SemaphoreType.DMA])
    def gather(table, idx2, out, idx_v, rows0, rows1, sem):
        c = lax.axis_index("c")
        s = lax.axis_index("s")
        wid = s * NC + c
        base = wid * EPW
        pltpu.sync_copy(idx2.at[pl.ds(wid * NCH, NCH)], idx_v)
        bufs = (rows0, rows1)
        cp = pltpu.async_copy(table.at[idx_v.at[0]], rows0, sem)
        for j in range(NCH):
            cp.wait()
            if j + 1 < NCH:
                cp = pltpu.async_copy(table.at[idx_v.at[j + 1]],
                                      bufs[(j + 1) % 2], sem)
            pltpu.sync_copy(bufs[j % 2], out.at[pl.ds(base + j * GCH, GCH)])

    @functools.partial(
        pl.kernel, mesh=mesh, compiler_params=params,
        out_type=(jax.ShapeDtypeStruct((NC * NP, HE), F32),
                  jax.ShapeDtypeStruct((NC * B, HE), F32)),
        scratch_types=[pltpu.VMEM_SHARED((NP, HE), F32),
                       pltpu.VMEM_SHARED((B, HE), F32),
                       pltpu.VMEM((GCH, HE), F32),
                       pltpu.VMEM((GCH, HE), F32),
                       pltpu.VMEM((NCH, GCH), jnp.int32),
                       pltpu.VMEM((NCH, GCH), jnp.int32),
                       pltpu.SemaphoreType.DMA])
    def edge_scatter(data, ridx2, eidx2, zn, ze, aggp, esump,
                     aggbuf, esbuf, d0, d1, ridx_v, eidx_v, sem):
        c = lax.axis_index("c")
        s = lax.axis_index("s")
        wid = s * NC + c
        rpw = NP // NS
        bpw = B // NS
        pltpu.sync_copy(zn.at[pl.ds(s * rpw, rpw)],
                        aggbuf.at[pl.ds(s * rpw, rpw)])
        pltpu.sync_copy(ze.at[pl.ds(s * bpw, bpw)],
                        esbuf.at[pl.ds(s * bpw, bpw)])
        pltpu.sync_copy(ridx2.at[pl.ds(wid * NCH, NCH)], ridx_v)
        pltpu.sync_copy(eidx2.at[pl.ds(wid * NCH, NCH)], eidx_v)
        plsc.subcore_barrier()
        base = wid * EPW
        bufs = (d0, d1)
        cp = pltpu.async_copy(data.at[pl.ds(base, GCH)], d0, sem)
        for j in range(NCH):
            cp.wait()
            if j + 1 < NCH:
                cp = pltpu.async_copy(data.at[pl.ds(base + (j + 1) * GCH, GCH)],
                                      bufs[(j + 1) % 2], sem)
            cur = bufs[j % 2]
            pltpu.sync_copy(cur, aggbuf.at[ridx_v.at[j]], add=True)
            pltpu.sync_copy(cur, esbuf.at[eidx_v.at[j]], add=True)
        plsc.subcore_barrier()
        pltpu.sync_copy(aggbuf.at[pl.ds(s * rpw, rpw)],
                        aggp.at[pl.ds(c * NP + s * rpw, rpw)])
        pltpu.sync_copy(esbuf.at[pl.ds(s * bpw, bpw)],
                        esump.at[pl.ds(c * B + s * bpw, bpw)])

    @functools.partial(
        pl.kernel, mesh=mesh, compiler_params=params,
        out_type=jax.ShapeDtypeStruct((NC * B, HE), F32),
        scratch_types=[pltpu.VMEM_SHARED((B, HE), F32),
                       pltpu.VMEM((GCH, HE), F32),
                       pltpu.VMEM((GCH, HE), F32),
                       pltpu.VMEM((NCH, GCH), jnp.int32),
                       pltpu.SemaphoreType.DMA])
    def eseg_scatter(data, eidx2, ze, esump, esbuf, d0, d1, eidx_v, sem):
        c = lax.axis_index("c")
        s = lax.axis_index("s")
        wid = s * NC + c
        bpw = B // NS
        pltpu.sync_copy(ze.at[pl.ds(s * bpw, bpw)],
                        esbuf.at[pl.ds(s * bpw, bpw)])
        pltpu.sync_copy(eidx2.at[pl.ds(wid * NCH, NCH)], eidx_v)
        plsc.subcore_barrier()
        base = wid * EPW
        bufs = (d0, d1)
        cp = pltpu.async_copy(data.at[pl.ds(base, GCH)], d0, sem)
        for j in range(NCH):
            cp.wait()
            if j + 1 < NCH:
                cp = pltpu.async_copy(data.at[pl.ds(base + (j + 1) * GCH, GCH)],
                                      bufs[(j + 1) % 2], sem)
            pltpu.sync_copy(bufs[j % 2], esbuf.at[eidx_v.at[j]], add=True)
        plsc.subcore_barrier()
        pltpu.sync_copy(esbuf.at[pl.ds(s * bpw, bpw)],
                        esump.at[pl.ds(c * B + s * bpw, bpw)])

    @functools.partial(
        pl.kernel, mesh=mesh, compiler_params=params,
        out_type=jax.ShapeDtypeStruct((NC * NSB, HN), F32),
        scratch_types=[pltpu.VMEM_SHARED((NSB, HN), F32),
                       pltpu.VMEM((NPW, HN), F32),
                       pltpu.VMEM((NPW,), jnp.int32)])
    def node_scatter(data, nidx, zns, nsump, nsbuf, data_v, nidx_v):
        c = lax.axis_index("c")
        s = lax.axis_index("s")
        rpw = NSB // NS
        pltpu.sync_copy(zns.at[pl.ds(s * rpw, rpw)],
                        nsbuf.at[pl.ds(s * rpw, rpw)])
        plsc.subcore_barrier()
        base = (s * NC + c) * NPW
        pltpu.sync_copy(data.at[pl.ds(base, NPW)], data_v)
        pltpu.sync_copy(nidx.at[pl.ds(base, NPW)], nidx_v)
        pltpu.sync_copy(data_v, nsbuf.at[nidx_v], add=True)
        plsc.subcore_barrier()
        pltpu.sync_copy(nsbuf.at[pl.ds(s * rpw, rpw)],
                        nsump.at[pl.ds(c * NSB + s * rpw, rpw)])

    @functools.partial(
        pl.kernel, mesh=mesh, compiler_params=params,
        out_type=(jax.ShapeDtypeStruct((NC * NP, 16), F32),
                  jax.ShapeDtypeStruct((NC * NSB, 16), F32),
                  jax.ShapeDtypeStruct((NC * B, 16), F32)),
        scratch_types=[pltpu.VMEM_SHARED((NP, 16), F32),
                       pltpu.VMEM_SHARED((NSB, 16), F32),
                       pltpu.VMEM_SHARED((B, 16), F32),
                       pltpu.VMEM((NPW, 16), F32),
                       pltpu.VMEM((NCH, GCH), jnp.int32),
                       pltpu.VMEM((NCH, GCH), jnp.int32),
                       pltpu.VMEM((NPW,), jnp.int32)])
    def counts(ridx2, eidx2, nidx, ones, zdeg, znc, zec, degp, ncp, ecp,
               degbuf, ncbuf, ecbuf, ones_v, ridx_v, eidx_v, nidx_v):
        c = lax.axis_index("c")
        s = lax.axis_index("s")
        wid = s * NC + c
        rpw = NP // NS
        npw2 = NSB // NS
        bpw = B // NS
        pltpu.sync_copy(zdeg.at[pl.ds(s * rpw, rpw)],
                        degbuf.at[pl.ds(s * rpw, rpw)])
        pltpu.sync_copy(znc.at[pl.ds(s * npw2, npw2)],
                        ncbuf.at[pl.ds(s * npw2, npw2)])
        pltpu.sync_copy(zec.at[pl.ds(s * bpw, bpw)],
                        ecbuf.at[pl.ds(s * bpw, bpw)])
        pltpu.sync_copy(ones, ones_v)
        pltpu.sync_copy(ridx2.at[pl.ds(wid * NCH, NCH)], ridx_v)
        pltpu.sync_copy(eidx2.at[pl.ds(wid * NCH, NCH)], eidx_v)
        plsc.subcore_barrier()
        for j in range(NCH):
            src = ones_v.at[pl.ds(0, GCH)]
            pltpu.sync_copy(src, degbuf.at[ridx_v.at[j]], add=True)
            pltpu.sync_copy(src, ecbuf.at[eidx_v.at[j]], add=True)
        nb = wid * NPW
        pltpu.sync_copy(nidx.at[pl.ds(nb, NPW)], nidx_v)
        pltpu.sync_copy(ones_v, ncbuf.at[nidx_v], add=True)
        plsc.subcore_barrier()
        pltpu.sync_copy(degbuf.at[pl.ds(s * rpw, rpw)],
                        degp.at[pl.ds(c * NP + s * rpw, rpw)])
        pltpu.sync_copy(ncbuf.at[pl.ds(s * npw2, npw2)],
                        ncp.at[pl.ds(c * NSB + s * npw2, npw2)])
        pltpu.sync_copy(ecbuf.at[pl.ds(s * bpw, bpw)],
                        ecp.at[pl.ds(c * B + s * bpw, bpw)])

    return types.SimpleNamespace(
        gather=gather, edge_scatter=edge_scatter, eseg_scatter=eseg_scatter,
        node_scatter=node_scatter, counts=counts)


def _sc_gather(table, idx):
    return _sc().gather(table, idx)


def _sc_edge_scatter(data, ridx, eidx, zn, ze):
    return _sc().edge_scatter(data, ridx, eidx, zn, ze)


def _sc_eseg_scatter(data, eidx, ze):
    return _sc().eseg_scatter(data, eidx, ze)


def _sc_node_scatter(data, nidx, zns):
    return _sc().node_scatter(data, nidx, zns)


def _sc_counts(ridx, eidx, nidx, ones, zdeg, znc, zec):
    return _sc().counts(ridx, eidx, nidx, ones, zdeg, znc, zec)


def _bdiag(w):
    z = jnp.zeros_like(w)
    return jnp.concatenate([jnp.concatenate([w, z], 1),
                            jnp.concatenate([z, w], 1)], 0)


def _btile(b):
    return jnp.concatenate([b, b]).reshape(1, -1)


def kernel(node_features, edge_features, senders, receivers, node_graph_ids,
           edge_graph_ids, enc_edge_W, enc_edge_b, enc_node_W, enc_node_b,
           enc_glob_Wn, enc_glob_We, enc_glob_b, hid_e_We, hid_e_Ws, hid_e_b,
           hid_n_Wn, hid_n_Win, hid_n_b, hid_g_Wn, hid_g_We, hid_g_Wg,
           hid_g_b, ro_W, ro_b):
    senders32 = senders.astype(jnp.int32)
    receivers32 = receivers.astype(jnp.int32)
    eidx32 = edge_graph_ids.astype(jnp.int32)
    nidpad = jnp.concatenate(
        [node_graph_ids.astype(jnp.int32),
         jnp.full((NP - N,), B, jnp.int32)])

    def perm(x):
        return jnp.stack([x[:EP], x[EP:]], axis=1).reshape(E)

    senders2 = (perm(senders32) * 2).reshape(E // GCH, GCH)
    ridx2 = perm(receivers32).reshape(E // GCH, GCH)
    eidx2 = perm(eidx32).reshape(E // GCH, GCH)

    eft = edge_features.T.reshape(12, EP)
    ztop = jnp.zeros_like(enc_edge_W)
    wE = jnp.stack([jnp.concatenate([enc_edge_W, ztop], 1),
                    jnp.concatenate([ztop, enc_edge_W], 1)], 1).reshape(
                        12, 2 * HE)
    nf = jnp.pad(node_features, ((0, NP - N), (0, 1)))
    wN = jnp.pad(enc_node_W, ((0, 1), (0, 0)))

    ones_c = jnp.ones((NPW, 16), F32)
    z_np64 = jnp.zeros((NP, HE), F32)
    z_b64 = jnp.zeros((B, HE), F32)
    z_ns = jnp.zeros((NSB, HN), F32)
    z_deg = jnp.zeros((NP, 16), F32)
    z_nc = jnp.zeros((NSB, 16), F32)
    z_ec = jnp.zeros((B, 16), F32)

    degp, ncp, ecp = _sc_counts(ridx2, eidx2, nidpad, ones_c,
                                z_deg, z_nc, z_ec)
    d0 = degp[:NP, 0:1]
    d1 = degp[NP:, 0:1]
    nc0 = ncp[:B, 0:1]
    nc1 = ncp[NSB:NSB + B, 0:1]
    ec0 = ecp[:B, 0:1]
    ec1 = ecp[B:, 0:1]

    e = _encT(eft, wE, _btile(enc_edge_b), 3200)
    n = _lin_relu(nf, wN, enc_node_b.reshape(1, HN), 2048)

    esump = _sc_eseg_scatter(e.reshape(E, HE), eidx2, z_b64)
    nsump = _sc_node_scatter(n, nidpad, z_ns)
    nps = [nsump]
    eps = [esump]

    for i in range(L):
        ns, nw = _nprep(n, hid_e_Ws[i], hid_n_Wn[i], 2048)
        gns = _sc_gather(ns.reshape(2 * NP, HE), senders2)
        e_new = _enew(e, gns.reshape(EP, 2 * HE), _bdiag(hid_e_We[i]),
                      _btile(hid_e_b[i]), 3200)
        aggp, esump = _sc_edge_scatter(e_new.reshape(E, HE), ridx2,
                                       eidx2, z_np64, z_b64)
        n_new = _nnew(nw, aggp[:NP], aggp[NP:], d0, d1,
                      hid_n_Win[i], hid_n_b[i].reshape(1, HN), 2048)
        nsump = _sc_node_scatter(n_new, nidpad, z_ns)
        nps.append(nsump)
        eps.append(esump)
        e, n = e_new, n_new

    np0 = jnp.stack([x[:B] for x in nps])
    np1 = jnp.stack([x[NSB:NSB + B] for x in nps])
    ep0 = jnp.stack([x[:B] for x in eps])
    ep1 = jnp.stack([x[B:2 * B] for x in eps])
    return _globchain(np0, np1, ep0, ep1, nc0, nc1, ec0, ec1,
                      enc_glob_Wn, enc_glob_We, enc_glob_b.reshape(1, HG),
                      hid_g_Wn, hid_g_We, hid_g_Wg, hid_g_b,
                      ro_W, ro_b.reshape(1, 1))


# speedup vs baseline: 1.0022x; 1.0022x over previous
import functools
import types

import jax
import jax.numpy as jnp
from jax import lax
from jax.experimental import pallas as pl
from jax.experimental.pallas import tpu as pltpu
from jax.experimental.pallas import tpu_sc as plsc

N = 10000
E = 160000
B = 512
HN = 256
HE = 64
HG = 32
L = 3

EP = E // 2
NP = 10240
NC = 2
NS = 16
NW = NC * NS
EPW = E // NW
GCH = 200
NCH = EPW // GCH
NPW = NP // NW
NSB = 640
F32 = jnp.float32


def _lin_relu_body(x, w, b, o):
    o[...] = jnp.maximum(
        jnp.dot(x[...], w[...], preferred_element_type=F32) + b[...], 0.0)


def _lin_relu(x, w, b, rows):
    m, k = x.shape
    n = w.shape[1]
    return pl.pallas_call(
        _lin_relu_body,
        grid=(m // rows,),
        in_specs=[pl.BlockSpec((rows, k), lambda i: (i, 0)),
                  pl.BlockSpec((k, n), lambda i: (0, 0)),
                  pl.BlockSpec((1, n), lambda i: (0, 0))],
        out_specs=pl.BlockSpec((rows, n), lambda i: (i, 0)),
        out_shape=jax.ShapeDtypeStruct((m, n), F32),
    )(x, w, b)


def _encT_body(x, w, b, o):
    o[...] = jnp.maximum(
        lax.dot_general(x[...], w[...], (((0,), (0,)), ((), ())),
                        preferred_element_type=F32) + b[...], 0.0)


def _encT(xt, w, b, rows):
    k, m = xt.shape
    n = w.shape[1]
    return pl.pallas_call(
        _encT_body,
        grid=(m // rows,),
        in_specs=[pl.BlockSpec((k, rows), lambda i: (0, i)),
                  pl.BlockSpec((k, n), lambda i: (0, 0)),
                  pl.BlockSpec((1, n), lambda i: (0, 0))],
        out_specs=pl.BlockSpec((rows, n), lambda i: (i, 0)),
        out_shape=jax.ShapeDtypeStruct((m, n), F32),
    )(xt, w, b)


def _nprep_body(x, ws, wn, ns_o, nw_o):
    xv = x[...]
    t = jnp.dot(xv, ws[...], preferred_element_type=F32)
    ns_o[...] = jnp.concatenate([t, t], axis=1)
    nw_o[...] = jnp.dot(xv, wn[...], preferred_element_type=F32)


def _nprep(nmat, ws, wn, rows):
    return pl.pallas_call(
        _nprep_body,
        grid=(NP // rows,),
        in_specs=[pl.BlockSpec((rows, HN), lambda i: (i, 0)),
                  pl.BlockSpec((HN, HE), lambda i: (0, 0)),
                  pl.BlockSpec((HN, HN), lambda i: (0, 0))],
        out_specs=[pl.BlockSpec((rows, 2 * HE), lambda i: (i, 0)),
                   pl.BlockSpec((rows, HN), lambda i: (i, 0))],
        out_shape=[jax.ShapeDtypeStruct((NP, 2 * HE), F32),
                   jax.ShapeDtypeStruct((NP, HN), F32)],
    )(nmat, ws, wn)


def _enew_body(e, g, w, b, o):
    o[...] = jnp.maximum(
        jnp.dot(e[...], w[...], preferred_element_type=F32) + g[...] + b[...],
        0.0)


def _enew(e, gns, w, b, rows):
    return pl.pallas_call(
        _enew_body,
        grid=(EP // rows,),
        in_specs=[pl.BlockSpec((rows, 2 * HE), lambda i: (i, 0)),
                  pl.BlockSpec((rows, 2 * HE), lambda i: (i, 0)),
                  pl.BlockSpec((2 * HE, 2 * HE), lambda i: (0, 0)),
                  pl.BlockSpec((1, 2 * HE), lambda i: (0, 0))],
        out_specs=pl.BlockSpec((rows, 2 * HE), lambda i: (i, 0)),
        out_shape=jax.ShapeDtypeStruct((EP, 2 * HE), F32),
    )(e, gns, w, b)


def _nnew_body(nw, a0, a1, d0, d1, win, b, o):
    agg = (a0[...] + a1[...]) / jnp.maximum(d0[...] + d1[...], 1.0)
    o[...] = jnp.maximum(
        nw[...] + jnp.dot(agg, win[...], preferred_element_type=F32) + b[...],
        0.0)


def _nnew(nw, a0, a1, d0, d1, win, b, rows):
    return pl.pallas_call(
        _nnew_body,
        grid=(NP // rows,),
        in_specs=[pl.BlockSpec((rows, HN), lambda i: (i, 0)),
                  pl.BlockSpec((rows, HE), lambda i: (i, 0)),
                  pl.BlockSpec((rows, HE), lambda i: (i, 0)),
                  pl.BlockSpec((rows, 1), lambda i: (i, 0)),
                  pl.BlockSpec((rows, 1), lambda i: (i, 0)),
                  pl.BlockSpec((HE, HN), lambda i: (0, 0)),
                  pl.BlockSpec((1, HN), lambda i: (0, 0))],
        out_specs=pl.BlockSpec((rows, HN), lambda i: (i, 0)),
        out_shape=jax.ShapeDtypeStruct((NP, HN), F32),
    )(nw, a0, a1, d0, d1, win, b)


def _glob_body(n0, n1, nc0, nc1, e0, e1, ec0, ec1, g, wn, we, wg, b, o):
    nmean = (n0[...] + n1[...]) / jnp.maximum(nc0[...] + nc1[...], 1.0)
    emean = (e0[...] + e1[...]) / jnp.maximum(ec0[...] + ec1[...], 1.0)
    o[...] = jnp.maximum(
        jnp.dot(nmean, wn[...], preferred_element_type=F32)
        + jnp.dot(emean, we[...], preferred_element_type=F32)
        + jnp.dot(g[...], wg[...], preferred_element_type=F32)
        + b[...], 0.0)


def _glob(n0, n1, nc0, nc1, e0, e1, ec0, ec1, g, wn, we, wg, b):
    return pl.pallas_call(
        _glob_body,
        out_shape=jax.ShapeDtypeStruct((B, HG), F32),
    )(n0, n1, nc0, nc1, e0, e1, ec0, ec1, g, wn, we, wg, b)


def _readout_body(g, w, b, o):
    o[...] = jnp.dot(g[...], w[...], preferred_element_type=F32) + b[...]


def _readout(g, w, b):
    return pl.pallas_call(
        _readout_body,
        out_shape=jax.ShapeDtypeStruct((B, 1), F32),
    )(g, w, b)


@functools.cache
def _sc():
    mesh = plsc.VectorSubcoreMesh(core_axis_name="c", subcore_axis_name="s",
                                  num_cores=NC, num_subcores=NS)
    params = pltpu.CompilerParams(use_tc_tiling_on_sc=False)

    @functools.partial(
        pl.kernel, mesh=mesh, compiler_params=params,
        out_type=jax.ShapeDtypeStruct((E, HE), F32),
        scratch_types=[pltpu.VMEM((NCH, GCH), jnp.int32),
                       pltpu.VMEM((GCH, HE), F32),
                       pltpu.VMEM((GCH, HE), F32),
                       pltpu.SemaphoreType.DMA])
    def gather(table, idx2, out, idx_v, rows0, rows1, sem):
        c = lax.axis_index("c")
        s = lax.axis_index("s")
        wid = s * NC + c
        base = wid * EPW
        pltpu.sync_copy(idx2.at[pl.ds(wid * NCH, NCH)], idx_v)
        bufs = (rows0, rows1)
        cp = pltpu.async_copy(table.at[idx_v.at[0]], rows0, sem)
        for j in range(NCH):
            cp.wait()
            if j + 1 < NCH:
                cp = pltpu.async_copy(table.at[idx_v.at[j + 1]],
                                      bufs[(j + 1) % 2], sem)
            pltpu.sync_copy(bufs[j % 2], out.at[pl.ds(base + j * GCH, GCH)])

    @functools.partial(
        pl.kernel, mesh=mesh, compiler_params=params,
        out_type=(jax.ShapeDtypeStruct((NC * NP, HE), F32),
                  jax.ShapeDtypeStruct((NC * B, HE), F32)),
        scratch_types=[pltpu.VMEM_SHARED((NP, HE), F32),
                       pltpu.VMEM_SHARED((B, HE), F32),
                       pltpu.VMEM((GCH, HE), F32),
                       pltpu.VMEM((GCH, HE), F32),
                       pltpu.VMEM((NCH, GCH), jnp.int32),
                       pltpu.VMEM((NCH, GCH), jnp.int32),
                       pltpu.SemaphoreType.DMA])
    def edge_scatter(data, ridx2, eidx2, zn, ze, aggp, esump,
                     aggbuf, esbuf, d0, d1, ridx_v, eidx_v, sem):
        c = lax.axis_index("c")
        s = lax.axis_index("s")
        wid = s * NC + c
        rpw = NP // NS
        bpw = B // NS
        pltpu.sync_copy(zn.at[pl.ds(s * rpw, rpw)],
                        aggbuf.at[pl.ds(s * rpw, rpw)])
        pltpu.sync_copy(ze.at[pl.ds(s * bpw, bpw)],
                        esbuf.at[pl.ds(s * bpw, bpw)])
        pltpu.sync_copy(ridx2.at[pl.ds(wid * NCH, NCH)], ridx_v)
        pltpu.sync_copy(eidx2.at[pl.ds(wid * NCH, NCH)], eidx_v)
        plsc.subcore_barrier()
        base = wid * EPW
        bufs = (d0, d1)
        cp = pltpu.async_copy(data.at[pl.ds(base, GCH)], d0, sem)
        for j in range(NCH):
            cp.wait()
            if j + 1 < NCH:
                cp = pltpu.async_copy(data.at[pl.ds(base + (j + 1) * GCH, GCH)],
                                      bufs[(j + 1) % 2], sem)
            cur = bufs[j % 2]
            pltpu.sync_copy(cur, aggbuf.at[ridx_v.at[j]], add=True)
            pltpu.sync_copy(cur, esbuf.at[eidx_v.at[j]], add=True)
        plsc.subcore_barrier()
        pltpu.sync_copy(aggbuf.at[pl.ds(s * rpw, rpw)],
                        aggp.at[pl.ds(c * NP + s * rpw, rpw)])
        pltpu.sync_copy(esbuf.at[pl.ds(s * bpw, bpw)],
                        esump.at[pl.ds(c * B + s * bpw, bpw)])

    @functools.partial(
        pl.kernel, mesh=mesh, compiler_params=params,
        out_type=jax.ShapeDtypeStruct((NC * B, HE), F32),
        scratch_types=[pltpu.VMEM_SHARED((B, HE), F32),
                       pltpu.VMEM((GCH, HE), F32),
                       pltpu.VMEM((GCH, HE), F32),
                       pltpu.VMEM((NCH, GCH), jnp.int32),
                       pltpu.SemaphoreType.DMA])
    def eseg_scatter(data, eidx2, ze, esump, esbuf, d0, d1, eidx_v, sem):
        c = lax.axis_index("c")
        s = lax.axis_index("s")
        wid = s * NC + c
        bpw = B // NS
        pltpu.sync_copy(ze.at[pl.ds(s * bpw, bpw)],
                        esbuf.at[pl.ds(s * bpw, bpw)])
        pltpu.sync_copy(eidx2.at[pl.ds(wid * NCH, NCH)], eidx_v)
        plsc.subcore_barrier()
        base = wid * EPW
        bufs = (d0, d1)
        cp = pltpu.async_copy(data.at[pl.ds(base, GCH)], d0, sem)
        for j in range(NCH):
            cp.wait()
            if j + 1 < NCH:
                cp = pltpu.async_copy(data.at[pl.ds(base + (j + 1) * GCH, GCH)],
                                      bufs[(j + 1) % 2], sem)
            pltpu.sync_copy(bufs[j % 2], esbuf.at[eidx_v.at[j]], add=True)
        plsc.subcore_barrier()
        pltpu.sync_copy(esbuf.at[pl.ds(s * bpw, bpw)],
                        esump.at[pl.ds(c * B + s * bpw, bpw)])

    @functools.partial(
        pl.kernel, mesh=mesh, compiler_params=params,
        out_type=jax.ShapeDtypeStruct((NC * NSB, HN), F32),
        scratch_types=[pltpu.VMEM_SHARED((NSB, HN), F32),
                       pltpu.VMEM((NPW, HN), F32),
                       pltpu.VMEM((NPW,), jnp.int32)])
    def node_scatter(data, nidx, zns, nsump, nsbuf, data_v, nidx_v):
        c = lax.axis_index("c")
        s = lax.axis_index("s")
        rpw = NSB // NS
        pltpu.sync_copy(zns.at[pl.ds(s * rpw, rpw)],
                        nsbuf.at[pl.ds(s * rpw, rpw)])
        plsc.subcore_barrier()
        base = (s * NC + c) * NPW
        pltpu.sync_copy(data.at[pl.ds(base, NPW)], data_v)
        pltpu.sync_copy(nidx.at[pl.ds(base, NPW)], nidx_v)
        pltpu.sync_copy(data_v, nsbuf.at[nidx_v], add=True)
        plsc.subcore_barrier()
        pltpu.sync_copy(nsbuf.at[pl.ds(s * rpw, rpw)],
                        nsump.at[pl.ds(c * NSB + s * rpw, rpw)])

    @functools.partial(
        pl.kernel, mesh=mesh, compiler_params=params,
        out_type=(jax.ShapeDtypeStruct((NC * NP, 16), F32),
                  jax.ShapeDtypeStruct((NC * NSB, 16), F32),
                  jax.ShapeDtypeStruct((NC * B, 16), F32)),
        scratch_types=[pltpu.VMEM_SHARED((NP, 16), F32),
                       pltpu.VMEM_SHARED((NSB, 16), F32),
                       pltpu.VMEM_SHARED((B, 16), F32),
                       pltpu.VMEM((NPW, 16), F32),
                       pltpu.VMEM((NCH, GCH), jnp.int32),
                       pltpu.VMEM((NCH, GCH), jnp.int32),
                       pltpu.VMEM((NPW,), jnp.int32)])
    def counts(ridx2, eidx2, nidx, ones, zdeg, znc, zec, degp, ncp, ecp,
               degbuf, ncbuf, ecbuf, ones_v, ridx_v, eidx_v, nidx_v):
        c = lax.axis_index("c")
        s = lax.axis_index("s")
        wid = s * NC + c
        rpw = NP // NS
        npw2 = NSB // NS
        bpw = B // NS
        pltpu.sync_copy(zdeg.at[pl.ds(s * rpw, rpw)],
                        degbuf.at[pl.ds(s * rpw, rpw)])
        pltpu.sync_copy(znc.at[pl.ds(s * npw2, npw2)],
                        ncbuf.at[pl.ds(s * npw2, npw2)])
        pltpu.sync_copy(zec.at[pl.ds(s * bpw, bpw)],
                        ecbuf.at[pl.ds(s * bpw, bpw)])
        pltpu.sync_copy(ones, ones_v)
        pltpu.sync_copy(ridx2.at[pl.ds(wid * NCH, NCH)], ridx_v)
        pltpu.sync_copy(eidx2.at[pl.ds(wid * NCH, NCH)], eidx_v)
        plsc.subcore_barrier()
        for j in range(NCH):
            src = ones_v.at[pl.ds(0, GCH)]
            pltpu.sync_copy(src, degbuf.at[ridx_v.at[j]], add=True)
            pltpu.sync_copy(src, ecbuf.at[eidx_v.at[j]], add=True)
        nb = wid * NPW
        pltpu.sync_copy(nidx.at[pl.ds(nb, NPW)], nidx_v)
        pltpu.sync_copy(ones_v, ncbuf.at[nidx_v], add=True)
        plsc.subcore_barrier()
        pltpu.sync_copy(degbuf.at[pl.ds(s * rpw, rpw)],
                        degp.at[pl.ds(c * NP + s * rpw, rpw)])
        pltpu.sync_copy(ncbuf.at[pl.ds(s * npw2, npw2)],
                        ncp.at[pl.ds(c * NSB + s * npw2, npw2)])
        pltpu.sync_copy(ecbuf.at[pl.ds(s * bpw, bpw)],
                        ecp.at[pl.ds(c * B + s * bpw, bpw)])

    return types.SimpleNamespace(
        gather=gather, edge_scatter=edge_scatter, eseg_scatter=eseg_scatter,
        node_scatter=node_scatter, counts=counts)


def _sc_gather(table, idx):
    return _sc().gather(table, idx)


def _sc_edge_scatter(data, ridx, eidx, zn, ze):
    return _sc().edge_scatter(data, ridx, eidx, zn, ze)


def _sc_eseg_scatter(data, eidx, ze):
    return _sc().eseg_scatter(data, eidx, ze)


def _sc_node_scatter(data, nidx, zns):
    return _sc().node_scatter(data, nidx, zns)


def _sc_counts(ridx, eidx, nidx, ones, zdeg, znc, zec):
    return _sc().counts(ridx, eidx, nidx, ones, zdeg, znc, zec)


def _bdiag(w):
    z = jnp.zeros_like(w)
    return jnp.concatenate([jnp.concatenate([w, z], 1),
                            jnp.concatenate([z, w], 1)], 0)


def _btile(b):
    return jnp.concatenate([b, b]).reshape(1, -1)


def kernel(node_features, edge_features, senders, receivers, node_graph_ids,
           edge_graph_ids, enc_edge_W, enc_edge_b, enc_node_W, enc_node_b,
           enc_glob_Wn, enc_glob_We, enc_glob_b, hid_e_We, hid_e_Ws, hid_e_b,
           hid_n_Wn, hid_n_Win, hid_n_b, hid_g_Wn, hid_g_We, hid_g_Wg,
           hid_g_b, ro_W, ro_b):
    senders32 = senders.astype(jnp.int32)
    receivers32 = receivers.astype(jnp.int32)
    eidx32 = edge_graph_ids.astype(jnp.int32)
    nidpad = jnp.concatenate(
        [node_graph_ids.astype(jnp.int32),
         jnp.full((NP - N,), B, jnp.int32)])

    def perm(x):
        return jnp.stack([x[:EP], x[EP:]], axis=1).reshape(E)

    senders2 = (perm(senders32) * 2).reshape(E // GCH, GCH)
    ridx2 = perm(receivers32).reshape(E // GCH, GCH)
    eidx2 = perm(eidx32).reshape(E // GCH, GCH)

    eft = edge_features.T.reshape(12, EP)
    ztop = jnp.zeros_like(enc_edge_W)
    wE = jnp.stack([jnp.concatenate([enc_edge_W, ztop], 1),
                    jnp.concatenate([ztop, enc_edge_W], 1)], 1).reshape(
                        12, 2 * HE)
    nf = jnp.pad(node_features, ((0, NP - N), (0, 1)))
    wN = jnp.pad(enc_node_W, ((0, 1), (0, 0)))

    ones_c = jnp.ones((NPW, 16), F32)
    z_np64 = jnp.zeros((NP, HE), F32)
    z_b64 = jnp.zeros((B, HE), F32)
    z_ns = jnp.zeros((NSB, HN), F32)
    z_deg = jnp.zeros((NP, 16), F32)
    z_nc = jnp.zeros((NSB, 16), F32)
    z_ec = jnp.zeros((B, 16), F32)

    degp, ncp, ecp = _sc_counts(ridx2, eidx2, nidpad, ones_c,
                                z_deg, z_nc, z_ec)
    d0 = degp[:NP, 0:1]
    d1 = degp[NP:, 0:1]
    nc0 = ncp[:B, 0:1]
    nc1 = ncp[NSB:NSB + B, 0:1]
    ec0 = ecp[:B, 0:1]
    ec1 = ecp[B:, 0:1]

    e = _encT(eft, wE, _btile(enc_edge_b), 3200)
    n = _lin_relu(nf, wN, enc_node_b.reshape(1, HN), 2048)

    esump = _sc_eseg_scatter(e.reshape(E, HE), eidx2, z_b64)
    nsump = _sc_node_scatter(n, nidpad, z_ns)
    g = _glob(nsump[:B], nsump[NSB:NSB + B], nc0, nc1,
              esump[:B], esump[B:], ec0, ec1,
              jnp.zeros((B, HG), F32), enc_glob_Wn, enc_glob_We,
              jnp.zeros((HG, HG), F32), enc_glob_b.reshape(1, HG))

    for i in range(L):
        ns, nw = _nprep(n, hid_e_Ws[i], hid_n_Wn[i], 2048)
        gns = _sc_gather(ns.reshape(2 * NP, HE), senders2)
        e_new = _enew(e, gns.reshape(EP, 2 * HE), _bdiag(hid_e_We[i]),
                      _btile(hid_e_b[i]), 3200)
        aggp, esump = _sc_edge_scatter(e_new.reshape(E, HE), ridx2,
                                       eidx2, z_np64, z_b64)
        n_new = _nnew(nw, aggp[:NP], aggp[NP:], d0, d1,
                      hid_n_Win[i], hid_n_b[i].reshape(1, HN), 2048)
        nsump = _sc_node_scatter(n_new, nidpad, z_ns)
        g = _glob(nsump[:B], nsump[NSB:NSB + B], nc0, nc1,
                  esump[:B], esump[B:], ec0, ec1,
                  g, hid_g_Wn[i], hid_g_We[i], hid_g_Wg[i],
                  hid_g_b[i].reshape(1, HG))
        e, n = e_new, n_new

    return _readout(g, ro_W, ro_b.reshape(1, 1))


# speedup vs baseline: 1.2012x; 1.1986x over previous
import functools
import types

import jax
import jax.numpy as jnp
from jax import lax
from jax.experimental import pallas as pl
from jax.experimental.pallas import tpu as pltpu
from jax.experimental.pallas import tpu_sc as plsc

N = 10000
E = 160000
B = 512
HN = 256
HE = 64
HG = 32
L = 3

EP = E // 2
NP = 10240
NC = 2
NS = 16
NW = NC * NS
EPW = E // NW
GCH = 200
NCH = EPW // GCH
NPW = NP // NW
NSB = 640
F32 = jnp.float32


def _lin_relu_body(x, w, b, o):
    o[...] = jnp.maximum(
        jnp.dot(x[...], w[...], preferred_element_type=F32) + b[...], 0.0)


def _lin_relu(x, w, b, rows):
    m, k = x.shape
    n = w.shape[1]
    return pl.pallas_call(
        _lin_relu_body,
        grid=(m // rows,),
        in_specs=[pl.BlockSpec((rows, k), lambda i: (i, 0)),
                  pl.BlockSpec((k, n), lambda i: (0, 0)),
                  pl.BlockSpec((1, n), lambda i: (0, 0))],
        out_specs=pl.BlockSpec((rows, n), lambda i: (i, 0)),
        out_shape=jax.ShapeDtypeStruct((m, n), F32),
    )(x, w, b)


def _encT_body(x, w, b, o):
    o[...] = jnp.maximum(
        lax.dot_general(x[...], w[...], (((0,), (0,)), ((), ())),
                        preferred_element_type=F32) + b[...], 0.0)


def _encT(xt, w, b, rows):
    k, m = xt.shape
    n = w.shape[1]
    return pl.pallas_call(
        _encT_body,
        grid=(m // rows,),
        in_specs=[pl.BlockSpec((k, rows), lambda i: (0, i)),
                  pl.BlockSpec((k, n), lambda i: (0, 0)),
                  pl.BlockSpec((1, n), lambda i: (0, 0))],
        out_specs=pl.BlockSpec((rows, n), lambda i: (i, 0)),
        out_shape=jax.ShapeDtypeStruct((m, n), F32),
    )(xt, w, b)


def _nprep_body(x, ws, wn, ns_o, nw_o):
    xv = x[...]
    t = jnp.dot(xv, ws[...], preferred_element_type=F32)
    ns_o[...] = jnp.concatenate([t, t], axis=1)
    nw_o[...] = jnp.dot(xv, wn[...], preferred_element_type=F32)


def _nprep(nmat, ws, wn, rows):
    return pl.pallas_call(
        _nprep_body,
        grid=(NP // rows,),
        in_specs=[pl.BlockSpec((rows, HN), lambda i: (i, 0)),
                  pl.BlockSpec((HN, HE), lambda i: (0, 0)),
                  pl.BlockSpec((HN, HN), lambda i: (0, 0))],
        out_specs=[pl.BlockSpec((rows, 2 * HE), lambda i: (i, 0)),
                   pl.BlockSpec((rows, HN), lambda i: (i, 0))],
        out_shape=[jax.ShapeDtypeStruct((NP, 2 * HE), F32),
                   jax.ShapeDtypeStruct((NP, HN), F32)],
    )(nmat, ws, wn)


def _enew_body(e, g, w, b, o):
    o[...] = jnp.maximum(
        jnp.dot(e[...], w[...], preferred_element_type=F32) + g[...] + b[...],
        0.0)


def _enew(e, gns, w, b, rows):
    return pl.pallas_call(
        _enew_body,
        grid=(EP // rows,),
        in_specs=[pl.BlockSpec((rows, 2 * HE), lambda i: (i, 0)),
                  pl.BlockSpec((rows, 2 * HE), lambda i: (i, 0)),
                  pl.BlockSpec((2 * HE, 2 * HE), lambda i: (0, 0)),
                  pl.BlockSpec((1, 2 * HE), lambda i: (0, 0))],
        out_specs=pl.BlockSpec((rows, 2 * HE), lambda i: (i, 0)),
        out_shape=jax.ShapeDtypeStruct((EP, 2 * HE), F32),
    )(e, gns, w, b)


def _nnew_body(nw, a0, a1, d0, d1, win, b, o):
    agg = (a0[...] + a1[...]) / jnp.maximum(d0[...] + d1[...], 1.0)
    o[...] = jnp.maximum(
        nw[...] + jnp.dot(agg, win[...], preferred_element_type=F32) + b[...],
        0.0)


def _nnew(nw, a0, a1, d0, d1, win, b, rows):
    return pl.pallas_call(
        _nnew_body,
        grid=(NP // rows,),
        in_specs=[pl.BlockSpec((rows, HN), lambda i: (i, 0)),
                  pl.BlockSpec((rows, HE), lambda i: (i, 0)),
                  pl.BlockSpec((rows, HE), lambda i: (i, 0)),
                  pl.BlockSpec((rows, 1), lambda i: (i, 0)),
                  pl.BlockSpec((rows, 1), lambda i: (i, 0)),
                  pl.BlockSpec((HE, HN), lambda i: (0, 0)),
                  pl.BlockSpec((1, HN), lambda i: (0, 0))],
        out_specs=pl.BlockSpec((rows, HN), lambda i: (i, 0)),
        out_shape=jax.ShapeDtypeStruct((NP, HN), F32),
    )(nw, a0, a1, d0, d1, win, b)


def _glob_body(n0, n1, nc0, nc1, e0, e1, ec0, ec1, g, wn, we, wg, b, o):
    nmean = (n0[...] + n1[...]) / jnp.maximum(nc0[...] + nc1[...], 1.0)
    emean = (e0[...] + e1[...]) / jnp.maximum(ec0[...] + ec1[...], 1.0)
    o[...] = jnp.maximum(
        jnp.dot(nmean, wn[...], preferred_element_type=F32)
        + jnp.dot(emean, we[...], preferred_element_type=F32)
        + jnp.dot(g[...], wg[...], preferred_element_type=F32)
        + b[...], 0.0)


def _glob(n0, n1, nc0, nc1, e0, e1, ec0, ec1, g, wn, we, wg, b):
    return pl.pallas_call(
        _glob_body,
        out_shape=jax.ShapeDtypeStruct((B, HG), F32),
    )(n0, n1, nc0, nc1, e0, e1, ec0, ec1, g, wn, we, wg, b)


def _readout_body(g, w, b, o):
    o[...] = jnp.dot(g[...], w[...], preferred_element_type=F32) + b[...]


def _readout(g, w, b):
    return pl.pallas_call(
        _readout_body,
        out_shape=jax.ShapeDtypeStruct((B, 1), F32),
    )(g, w, b)


@functools.cache
def _sc():
    mesh = plsc.VectorSubcoreMesh(core_axis_name="c", subcore_axis_name="s",
                                  num_cores=NC, num_subcores=NS)
    params = pltpu.CompilerParams(use_tc_tiling_on_sc=False)

    @functools.partial(
        pl.kernel, mesh=mesh, compiler_params=params,
        out_type=jax.ShapeDtypeStruct((EP, 2 * HE), F32),
        scratch_types=[pltpu.VMEM((NCH, GCH), jnp.int32),
                       pltpu.VMEM((GCH, HE), F32),
                       pltpu.VMEM((GCH, HE), F32),
                       pltpu.SemaphoreType.DMA])
    def gather(table, idx2, out, idx_v, rows0, rows1, sem):
        c = lax.axis_index("c")
        s = lax.axis_index("s")
        wid = s * NC + c
        rbase = (wid % NS) * EPW
        coff = (wid // NS) * HE
        pltpu.sync_copy(idx2.at[pl.ds(wid * NCH, NCH)], idx_v)
        bufs = (rows0, rows1)
        cp = pltpu.async_copy(table.at[idx_v.at[0]], rows0, sem)
        for j in range(NCH):
            cp.wait()
            if j + 1 < NCH:
                cp = pltpu.async_copy(table.at[idx_v.at[j + 1]],
                                      bufs[(j + 1) % 2], sem)
            pltpu.sync_copy(bufs[j % 2],
                            out.at[pl.ds(rbase + j * GCH, GCH),
                                   pl.ds(coff, HE)])

    @functools.partial(
        pl.kernel, mesh=mesh, compiler_params=params,
        out_type=(jax.ShapeDtypeStruct((NC * NP, HE), F32),
                  jax.ShapeDtypeStruct((NC * B, HE), F32)),
        scratch_types=[pltpu.VMEM_SHARED((NP, HE), F32),
                       pltpu.VMEM_SHARED((B, HE), F32),
                       pltpu.VMEM((GCH, HE), F32),
                       pltpu.VMEM((GCH, HE), F32),
                       pltpu.VMEM((NCH, GCH), jnp.int32),
                       pltpu.VMEM((NCH, GCH), jnp.int32),
                       pltpu.SemaphoreType.DMA])
    def edge_scatter(data, ridx2, eidx2, zn, ze, aggp, esump,
                     aggbuf, esbuf, d0, d1, ridx_v, eidx_v, sem):
        c = lax.axis_index("c")
        s = lax.axis_index("s")
        wid = s * NC + c
        rbase = (wid % NS) * EPW
        coff = (wid // NS) * HE
        rpw = NP // NS
        bpw = B // NS
        pltpu.sync_copy(zn.at[pl.ds(s * rpw, rpw)],
                        aggbuf.at[pl.ds(s * rpw, rpw)])
        pltpu.sync_copy(ze.at[pl.ds(s * bpw, bpw)],
                        esbuf.at[pl.ds(s * bpw, bpw)])
        pltpu.sync_copy(ridx2.at[pl.ds(wid * NCH, NCH)], ridx_v)
        pltpu.sync_copy(eidx2.at[pl.ds(wid * NCH, NCH)], eidx_v)
        plsc.subcore_barrier()
        bufs = (d0, d1)
        cp = pltpu.async_copy(
            data.at[pl.ds(rbase, GCH), pl.ds(coff, HE)], d0, sem)
        for j in range(NCH):
            cp.wait()
            if j + 1 < NCH:
                cp = pltpu.async_copy(
                    data.at[pl.ds(rbase + (j + 1) * GCH, GCH),
                            pl.ds(coff, HE)], bufs[(j + 1) % 2], sem)
            cur = bufs[j % 2]
            pltpu.sync_copy(cur, aggbuf.at[ridx_v.at[j]], add=True)
            pltpu.sync_copy(cur, esbuf.at[eidx_v.at[j]], add=True)
        plsc.subcore_barrier()
        pltpu.sync_copy(aggbuf.at[pl.ds(s * rpw, rpw)],
                        aggp.at[pl.ds(c * NP + s * rpw, rpw)])
        pltpu.sync_copy(esbuf.at[pl.ds(s * bpw, bpw)],
                        esump.at[pl.ds(c * B + s * bpw, bpw)])

    @functools.partial(
        pl.kernel, mesh=mesh, compiler_params=params,
        out_type=jax.ShapeDtypeStruct((NC * B, HE), F32),
        scratch_types=[pltpu.VMEM_SHARED((B, HE), F32),
                       pltpu.VMEM((GCH, HE), F32),
                       pltpu.VMEM((GCH, HE), F32),
                       pltpu.VMEM((NCH, GCH), jnp.int32),
                       pltpu.SemaphoreType.DMA])
    def eseg_scatter(data, eidx2, ze, esump, esbuf, d0, d1, eidx_v, sem):
        c = lax.axis_index("c")
        s = lax.axis_index("s")
        wid = s * NC + c
        rbase = (wid % NS) * EPW
        coff = (wid // NS) * HE
        bpw = B // NS
        pltpu.sync_copy(ze.at[pl.ds(s * bpw, bpw)],
                        esbuf.at[pl.ds(s * bpw, bpw)])
        pltpu.sync_copy(eidx2.at[pl.ds(wid * NCH, NCH)], eidx_v)
        plsc.subcore_barrier()
        bufs = (d0, d1)
        cp = pltpu.async_copy(
            data.at[pl.ds(rbase, GCH), pl.ds(coff, HE)], d0, sem)
        for j in range(NCH):
            cp.wait()
            if j + 1 < NCH:
                cp = pltpu.async_copy(
                    data.at[pl.ds(rbase + (j + 1) * GCH, GCH),
                            pl.ds(coff, HE)], bufs[(j + 1) % 2], sem)
            pltpu.sync_copy(bufs[j % 2], esbuf.at[eidx_v.at[j]], add=True)
        plsc.subcore_barrier()
        pltpu.sync_copy(esbuf.at[pl.ds(s * bpw, bpw)],
                        esump.at[pl.ds(c * B + s * bpw, bpw)])

    @functools.partial(
        pl.kernel, mesh=mesh, compiler_params=params,
        out_type=jax.ShapeDtypeStruct((NC * NSB, HN), F32),
        scratch_types=[pltpu.VMEM_SHARED((NSB, HN), F32),
                       pltpu.VMEM((NPW, HN), F32),
                       pltpu.VMEM((NPW,), jnp.int32)])
    def node_scatter(data, nidx, zns, nsump, nsbuf, data_v, nidx_v):
        c = lax.axis_index("c")
        s = lax.axis_index("s")
        rpw = NSB // NS
        pltpu.sync_copy(zns.at[pl.ds(s * rpw, rpw)],
                        nsbuf.at[pl.ds(s * rpw, rpw)])
        plsc.subcore_barrier()
        base = (s * NC + c) * NPW
        pltpu.sync_copy(data.at[pl.ds(base, NPW)], data_v)
        pltpu.sync_copy(nidx.at[pl.ds(base, NPW)], nidx_v)
        pltpu.sync_copy(data_v, nsbuf.at[nidx_v], add=True)
        plsc.subcore_barrier()
        pltpu.sync_copy(nsbuf.at[pl.ds(s * rpw, rpw)],
                        nsump.at[pl.ds(c * NSB + s * rpw, rpw)])

    @functools.partial(
        pl.kernel, mesh=mesh, compiler_params=params,
        out_type=(jax.ShapeDtypeStruct((NC * NP, 16), F32),
                  jax.ShapeDtypeStruct((NC * NSB, 16), F32),
                  jax.ShapeDtypeStruct((NC * B, 16), F32)),
        scratch_types=[pltpu.VMEM_SHARED((NP, 16), F32),
                       pltpu.VMEM_SHARED((NSB, 16), F32),
                       pltpu.VMEM_SHARED((B, 16), F32),
                       pltpu.VMEM((NPW, 16), F32),
                       pltpu.VMEM((NCH, GCH), jnp.int32),
                       pltpu.VMEM((NCH, GCH), jnp.int32),
                       pltpu.VMEM((NPW,), jnp.int32)])
    def counts(ridx2, eidx2, nidx, ones, zdeg, znc, zec, degp, ncp, ecp,
               degbuf, ncbuf, ecbuf, ones_v, ridx_v, eidx_v, nidx_v):
        c = lax.axis_index("c")
        s = lax.axis_index("s")
        wid = s * NC + c
        rpw = NP // NS
        npw2 = NSB // NS
        bpw = B // NS
        pltpu.sync_copy(zdeg.at[pl.ds(s * rpw, rpw)],
                        degbuf.at[pl.ds(s * rpw, rpw)])
        pltpu.sync_copy(znc.at[pl.ds(s * npw2, npw2)],
                        ncbuf.at[pl.ds(s * npw2, npw2)])
        pltpu.sync_copy(zec.at[pl.ds(s * bpw, bpw)],
                        ecbuf.at[pl.ds(s * bpw, bpw)])
        pltpu.sync_copy(ones, ones_v)
        pltpu.sync_copy(ridx2.at[pl.ds(wid * NCH, NCH)], ridx_v)
        pltpu.sync_copy(eidx2.at[pl.ds(wid * NCH, NCH)], eidx_v)
        plsc.subcore_barrier()
        for j in range(NCH):
            src = ones_v.at[pl.ds(0, GCH)]
            pltpu.sync_copy(src, degbuf.at[ridx_v.at[j]], add=True)
            pltpu.sync_copy(src, ecbuf.at[eidx_v.at[j]], add=True)
        nb = wid * NPW
        pltpu.sync_copy(nidx.at[pl.ds(nb, NPW)], nidx_v)
        pltpu.sync_copy(ones_v, ncbuf.at[nidx_v], add=True)
        plsc.subcore_barrier()
        pltpu.sync_copy(degbuf.at[pl.ds(s * rpw, rpw)],
                        degp.at[pl.ds(c * NP + s * rpw, rpw)])
        pltpu.sync_copy(ncbuf.at[pl.ds(s * npw2, npw2)],
                        ncp.at[pl.ds(c * NSB + s * npw2, npw2)])
        pltpu.sync_copy(ecbuf.at[pl.ds(s * bpw, bpw)],
                        ecp.at[pl.ds(c * B + s * bpw, bpw)])

    return types.SimpleNamespace(
        gather=gather, edge_scatter=edge_scatter, eseg_scatter=eseg_scatter,
        node_scatter=node_scatter, counts=counts)


def _sc_gather(table, idx):
    return _sc().gather(table, idx)


def _sc_edge_scatter(data, ridx, eidx, zn, ze):
    return _sc().edge_scatter(data, ridx, eidx, zn, ze)


def _sc_eseg_scatter(data, eidx, ze):
    return _sc().eseg_scatter(data, eidx, ze)


def _sc_node_scatter(data, nidx, zns):
    return _sc().node_scatter(data, nidx, zns)


def _sc_counts(ridx, eidx, nidx, ones, zdeg, znc, zec):
    return _sc().counts(ridx, eidx, nidx, ones, zdeg, znc, zec)


def _bdiag(w):
    z = jnp.zeros_like(w)
    return jnp.concatenate([jnp.concatenate([w, z], 1),
                            jnp.concatenate([z, w], 1)], 0)


def _btile(b):
    return jnp.concatenate([b, b]).reshape(1, -1)


def kernel(node_features, edge_features, senders, receivers, node_graph_ids,
           edge_graph_ids, enc_edge_W, enc_edge_b, enc_node_W, enc_node_b,
           enc_glob_Wn, enc_glob_We, enc_glob_b, hid_e_We, hid_e_Ws, hid_e_b,
           hid_n_Wn, hid_n_Win, hid_n_b, hid_g_Wn, hid_g_We, hid_g_Wg,
           hid_g_b, ro_W, ro_b):
    senders32 = senders.astype(jnp.int32)
    receivers32 = receivers.astype(jnp.int32)
    eidx32 = edge_graph_ids.astype(jnp.int32)
    nidpad = jnp.concatenate(
        [node_graph_ids.astype(jnp.int32),
         jnp.full((NP - N,), B, jnp.int32)])

    senders2 = (senders32 * 2).reshape(E // GCH, GCH)
    ridx2 = receivers32.reshape(E // GCH, GCH)
    eidx2 = eidx32.reshape(E // GCH, GCH)

    eft = edge_features.T.reshape(12, EP)
    ztop = jnp.zeros_like(enc_edge_W)
    wE = jnp.stack([jnp.concatenate([enc_edge_W, ztop], 1),
                    jnp.concatenate([ztop, enc_edge_W], 1)], 1).reshape(
                        12, 2 * HE)
    nf = jnp.pad(node_features, ((0, NP - N), (0, 1)))
    wN = jnp.pad(enc_node_W, ((0, 1), (0, 0)))

    ones_c = jnp.ones((NPW, 16), F32)
    z_np64 = jnp.zeros((NP, HE), F32)
    z_b64 = jnp.zeros((B, HE), F32)
    z_ns = jnp.zeros((NSB, HN), F32)
    z_deg = jnp.zeros((NP, 16), F32)
    z_nc = jnp.zeros((NSB, 16), F32)
    z_ec = jnp.zeros((B, 16), F32)

    degp, ncp, ecp = _sc_counts(ridx2, eidx2, nidpad, ones_c,
                                z_deg, z_nc, z_ec)
    d0 = degp[:NP, 0:1]
    d1 = degp[NP:, 0:1]
    nc0 = ncp[:B, 0:1]
    nc1 = ncp[NSB:NSB + B, 0:1]
    ec0 = ecp[:B, 0:1]
    ec1 = ecp[B:, 0:1]

    e = _encT(eft, wE, _btile(enc_edge_b), 3200)
    n = _lin_relu(nf, wN, enc_node_b.reshape(1, HN), 2048)

    esump = _sc_eseg_scatter(e, eidx2, z_b64)
    nsump = _sc_node_scatter(n, nidpad, z_ns)
    g = _glob(nsump[:B], nsump[NSB:NSB + B], nc0, nc1,
              esump[:B], esump[B:], ec0, ec1,
              jnp.zeros((B, HG), F32), enc_glob_Wn, enc_glob_We,
              jnp.zeros((HG, HG), F32), enc_glob_b.reshape(1, HG))

    for i in range(L):
        ns, nw = _nprep(n, hid_e_Ws[i], hid_n_Wn[i], 2048)
        gns = _sc_gather(ns.reshape(2 * NP, HE), senders2)
        e_new = _enew(e, gns, _bdiag(hid_e_We[i]),
                      _btile(hid_e_b[i]), 3200)
        aggp, esump = _sc_edge_scatter(e_new, ridx2, eidx2, z_np64, z_b64)
        n_new = _nnew(nw, aggp[:NP], aggp[NP:], d0, d1,
                      hid_n_Win[i], hid_n_b[i].reshape(1, HN), 2048)
        nsump = _sc_node_scatter(n_new, nidpad, z_ns)
        g = _glob(nsump[:B], nsump[NSB:NSB + B], nc0, nc1,
                  esump[:B], esump[B:], ec0, ec1,
                  g, hid_g_Wn[i], hid_g_We[i], hid_g_Wg[i],
                  hid_g_b[i].reshape(1, HG))
        e, n = e_new, n_new

    return _readout(g, ro_W, ro_b.reshape(1, 1))
